# baseline (speedup 1.0000x reference)
import functools

import jax
import jax.numpy as jnp
from jax import lax
from jax.experimental import pallas as pl
from jax.experimental.pallas import tpu as pltpu
from jax.experimental.pallas import tpu_sc as plsc

_N_USERS = 25000
_NN = 50000
_E = 800000
_HALF = 32
_NS = 16
_NC = 2
_CHUNK = 64
_SBC = 8
_NSB = 100
_CPT = _NSB * _SBC
_NROWS = _NS * _CPT
_EPAD = _NROWS * _CHUNK
_RPT = 3128
_ZROWS = 64
_NZCP = 49


def _hop(table, col2d, row2d, vals2d):
    mesh = plsc.VectorSubcoreMesh(core_axis_name="c", subcore_axis_name="s")

    @functools.partial(
        pl.kernel,
        out_type=jax.ShapeDtypeStruct((_NC, _NN, _HALF), jnp.float32),
        mesh=mesh,
        scratch_types=[
            pltpu.VMEM_SHARED((_NN, _HALF), jnp.float32),
            pltpu.VMEM((2, _SBC, _CHUNK), jnp.int32),
            pltpu.VMEM((2, _SBC, _CHUNK), jnp.int32),
            pltpu.VMEM((2, _SBC, _CHUNK), jnp.float32),
            pltpu.VMEM((8, _CHUNK, _HALF), jnp.float32),
            pltpu.VMEM((_ZROWS, _HALF), jnp.float32),
            [pltpu.SemaphoreType.DMA] * 8,
            [pltpu.SemaphoreType.DMA] * 8,
            [pltpu.SemaphoreType.DMA] * 2,
        ],
        compiler_params=pltpu.CompilerParams(use_tc_tiling_on_sc=False),
    )
    def hop_kernel(table_h, col_h, row_h, vals_h, out_h,
                   acc, colsb, rowsb, valssb, rowsbuf, zbuf,
                   gsem, ssem, bsem):
        cx = lax.axis_index("c")
        s = lax.axis_index("s")
        zero16 = jnp.zeros((16,), jnp.float32)

        def fill_zero(i, carry):
            zbuf[i, pl.ds(0, 16)] = zero16
            zbuf[i, pl.ds(16, 16)] = zero16
            return carry
        lax.fori_loop(0, _ZROWS, fill_zero, 0)

        tstart = jnp.minimum(s * _RPT, _NN - _RPT)

        def zero_acc(z, carry):
            start = jnp.minimum(z * _ZROWS, _RPT - _ZROWS) + tstart
            pltpu.sync_copy(zbuf, acc.at[pl.ds(start, _ZROWS), :])
            return carry
        lax.fori_loop(0, _NZCP, zero_acc, 0)
        plsc.subcore_barrier()

        tbase = s * _CPT

        def sb_copies(sbi, par):
            src = pl.ds(tbase + sbi * _SBC, _SBC)
            return (
                pltpu.make_async_copy(col_h.at[src, :], colsb.at[par],
                                      bsem[par]),
                pltpu.make_async_copy(row_h.at[src, :], rowsb.at[par],
                                      bsem[par]),
                pltpu.make_async_copy(vals_h.at[src, :], valssb.at[par],
                                      bsem[par]),
            )

        def gather(par, j, buf):
            return pltpu.make_async_copy(
                table_h.at[cx].at[colsb.at[par, j]], rowsbuf.at[buf],
                gsem[buf])

        def scatter(par, j, buf):
            return pltpu.make_async_copy(
                rowsbuf.at[buf], acc.at[rowsb.at[par, j]], ssem[buf])

        for d in sb_copies(0, 0):
            d.start()
            d.wait()
        gather(0, 0, 0).start()
        gather(0, 1, 1).start()
        gather(0, 2, 2).start()

        def sb_pair(ss, carry):
            for p in (0, 1):
                sbi = 2 * ss + p
                for j in range(_SBC):
                    gb = (j + 3) % 8
                    if j >= 5:
                        wdesc = scatter(p, j - 5, gb)
                    else:
                        wdesc = scatter(1 - p, j + 3, gb)
                    if p == 0 and j < 5:
                        @pl.when(ss > 0)
                        def _():
                            wdesc.wait()
                    else:
                        wdesc.wait()
                    if j == 4:
                        descs = sb_copies(sbi + 1, 1 - p)
                        if p == 1:
                            @pl.when(ss < _NSB // 2 - 1)
                            def _():
                                for d in descs:
                                    d.start()
                        else:
                            for d in descs:
                                d.start()
                    if j == 5:
                        descs = sb_copies(sbi + 1, 1 - p)
                        if p == 1:
                            @pl.when(ss < _NSB // 2 - 1)
                            def _():
                                for d in descs:
                                    d.wait()
                        else:
                            for d in descs:
                                d.wait()
                    if j < 5:
                        gdesc = gather(p, j + 3, gb)
                    else:
                        gdesc = gather(1 - p, j - 5, gb)
                    if p == 1 and j >= 5:
                        @pl.when(ss < _NSB // 2 - 1)
                        def _():
                            gdesc.start()
                    else:
                        gdesc.start()
                    gather(p, j, j).wait()

                    def scale(t, inner):
                        v16 = valssb[p, j, pl.ds(t * 16, 16)]
                        for i in range(16):
                            vv = jnp.take_along_axis(
                                v16, jnp.full((16,), i, jnp.int32), axis=0,
                                mode="promise_in_bounds")
                            e = t * 16 + i
                            for k in range(_HALF // 16):
                                rowsbuf[j, e, pl.ds(k * 16, 16)] = (
                                    rowsbuf[j, e, pl.ds(k * 16, 16)] * vv)
                        return inner
                    lax.fori_loop(0, _CHUNK // 16, scale, 0)

                    scatter(p, j, j).start(add=True)
            return carry
        lax.fori_loop(0, _NSB // 2, sb_pair, 0)

        for j in range(3, 8):
            scatter(1, j, j).wait()
        plsc.subcore_barrier()

        pltpu.sync_copy(acc.at[pl.ds(tstart, _RPT), :],
                        out_h.at[cx].at[pl.ds(tstart, _RPT), :])

    return hop_kernel(table, col2d, row2d, vals2d)


def kernel(user_embed, item_embed, edge_index, edge_vals):
    all_embed = jnp.concatenate([user_embed, item_embed], axis=0)
    table = jnp.stack([all_embed[:, :_HALF], all_embed[:, _HALF:]], axis=0)

    pad = _EPAD - _E
    colp = jnp.pad(edge_index[1], (0, pad)).reshape(_NROWS, _CHUNK)
    rowp = jnp.pad(edge_index[0], (0, pad)).reshape(_NROWS, _CHUNK)
    valsp = jnp.pad(edge_vals, (0, pad)).reshape(_NROWS, _CHUNK)

    embs = [all_embed]
    for _ in range(3):
        table = _hop(table, colp, rowp, valsp)
        embs.append(jnp.concatenate([table[0], table[1]], axis=1))
    embs = jnp.stack(embs, axis=1)
    return embs[:_N_USERS], embs[_N_USERS:]


# speedup vs baseline: 1.2212x; 1.2212x over previous
import functools

import jax
import jax.numpy as jnp
from jax import lax
from jax.experimental import pallas as pl
from jax.experimental.pallas import tpu as pltpu
from jax.experimental.pallas import tpu_sc as plsc

_N_USERS = 25000
_NN = 50000
_E = 800000
_HALF = 32
_NS = 16
_NC = 2
_NHOPS = 3
_CHUNK = 128
_SBC = 8
_NSB = 50
_CPT = _NSB * _SBC
_NROWS = _NS * _CPT
_EPAD = _NROWS * _CHUNK
_RPT = 3128
_UPT = 3128
_ZROWS = 64
_NZCP = 49
_LASTSS = _NSB // 2 - 1


def _propagate(table, col2d, row2d, vals2d):
    mesh = plsc.VectorSubcoreMesh(core_axis_name="c", subcore_axis_name="s")

    @functools.partial(
        pl.kernel,
        out_type=(
            jax.ShapeDtypeStruct((_N_USERS, _NHOPS, 2 * _HALF), jnp.float32),
            jax.ShapeDtypeStruct((_NN - _N_USERS, _NHOPS, 2 * _HALF),
                                 jnp.float32),
            jax.ShapeDtypeStruct((_NC, _NN, _HALF), jnp.float32),
        ),
        mesh=mesh,
        scratch_types=[
            pltpu.VMEM_SHARED((_NN, _HALF), jnp.float32),
            pltpu.VMEM((2, _SBC, _CHUNK), jnp.int32),
            pltpu.VMEM((2, _SBC, _CHUNK), jnp.int32),
            pltpu.VMEM((2, _SBC, _CHUNK), jnp.float32),
            pltpu.VMEM((4, _CHUNK, _HALF), jnp.float32),
            pltpu.VMEM((_ZROWS, _HALF), jnp.float32),
            [pltpu.SemaphoreType.DMA] * 4,
            [pltpu.SemaphoreType.DMA] * 4,
            [pltpu.SemaphoreType.DMA] * 2,
            [pltpu.SemaphoreType.DMA] * 2,
        ],
        compiler_params=pltpu.CompilerParams(use_tc_tiling_on_sc=False),
    )
    def prop_kernel(table_h, col_h, row_h, vals_h, users_h, items_h, stage_h,
                    acc, colsb, rowsb, valssb, rowsbuf, zbuf,
                    gsem, ssem, bsem, dsem):
        cx = lax.axis_index("c")
        s = lax.axis_index("s")
        zero16 = jnp.zeros((16,), jnp.float32)

        def fill_zero(i, carry):
            zbuf[i, pl.ds(0, 16)] = zero16
            zbuf[i, pl.ds(16, 16)] = zero16
            return carry
        lax.fori_loop(0, _ZROWS, fill_zero, 0)

        tstart = jnp.minimum(s * _RPT, _NN - _RPT)

        def zero_acc():
            def body(z, carry):
                start = jnp.minimum(z * _ZROWS, _RPT - _ZROWS) + tstart
                pltpu.sync_copy(zbuf, acc.at[pl.ds(start, _ZROWS), :])
                return carry
            lax.fori_loop(0, _NZCP, body, 0)

        tbase = s * _CPT

        def sb_copies(sbi, par):
            src = pl.ds(tbase + sbi * _SBC, _SBC)
            return (
                pltpu.make_async_copy(col_h.at[src, :], colsb.at[par],
                                      bsem[par]),
                pltpu.make_async_copy(row_h.at[src, :], rowsb.at[par],
                                      bsem[par]),
                pltpu.make_async_copy(vals_h.at[src, :], valssb.at[par],
                                      bsem[par]),
            )

        def scatter(par, j, buf):
            return pltpu.make_async_copy(
                rowsbuf.at[buf], acc.at[rowsb.at[par, j]], ssem[buf])

        def hop(table_ref, h):
            def gather(par, j, buf):
                return pltpu.make_async_copy(
                    table_ref.at[cx].at[colsb.at[par, j]], rowsbuf.at[buf],
                    gsem[buf])

            for d in sb_copies(0, 0):
                d.start()
                d.wait()
            gather(0, 0, 0).start()
            gather(0, 1, 1).start()

            def sb_pair(ss, carry):
                for p in (0, 1):
                    sbi = 2 * ss + p
                    for j in range(_SBC):
                        b = j % 4
                        wb = (j + 2) % 4
                        if j >= 2:
                            wdesc = scatter(p, j - 2, wb)
                        else:
                            wdesc = scatter(1 - p, j + 6, wb)
                        if p == 0 and j < 2:
                            @pl.when(ss > 0)
                            def _():
                                wdesc.wait()
                        else:
                            wdesc.wait()
                        if j == 2:
                            descs = sb_copies(sbi + 1, 1 - p)
                            if p == 1:
                                @pl.when(ss < _LASTSS)
                                def _():
                                    for d in descs:
                                        d.start()
                            else:
                                for d in descs:
                                    d.start()
                        if j == 6:
                            descs = sb_copies(sbi + 1, 1 - p)
                            if p == 1:
                                @pl.when(ss < _LASTSS)
                                def _():
                                    for d in descs:
                                        d.wait()
                            else:
                                for d in descs:
                                    d.wait()
                        if j < 6:
                            gdesc = gather(p, j + 2, wb)
                        else:
                            gdesc = gather(1 - p, j - 6, wb)
                        if p == 1 and j >= 6:
                            @pl.when(ss < _LASTSS)
                            def _():
                                gdesc.start()
                        else:
                            gdesc.start()
                        gather(p, j, b).wait()

                        def scale(t, inner):
                            v16 = valssb[p, j, pl.ds(t * 16, 16)]
                            for i in range(16):
                                vv = jnp.take_along_axis(
                                    v16, jnp.full((16,), i, jnp.int32),
                                    axis=0, mode="promise_in_bounds")
                                e = t * 16 + i
                                for k in range(_HALF // 16):
                                    rowsbuf[b, e, pl.ds(k * 16, 16)] = (
                                        rowsbuf[b, e, pl.ds(k * 16, 16)]
                                        * vv)
                            return inner
                        lax.fori_loop(0, _CHUNK // 16, scale, 0)

                        scatter(p, j, b).start(add=True)
                return carry
            lax.fori_loop(0, _NSB // 2, sb_pair, 0)

            scatter(1, 6, 2).wait()
            scatter(1, 7, 3).wait()
            plsc.subcore_barrier()

            descs = []
            if h < _NHOPS - 1:
                descs.append(pltpu.make_async_copy(
                    acc.at[pl.ds(tstart, _RPT), :],
                    stage_h.at[cx].at[pl.ds(tstart, _RPT), :], dsem[0]))

            @pl.when(s < _NS // 2)
            def _():
                ustart = jnp.minimum(s * _UPT, _N_USERS - _UPT)
                pltpu.async_copy(
                    acc.at[pl.ds(ustart, _UPT), :],
                    users_h.at[pl.ds(ustart, _UPT), h,
                               pl.ds(cx * _HALF, _HALF)], dsem[1])

            @pl.when(s >= _NS // 2)
            def _():
                istart = jnp.minimum((s - _NS // 2) * _UPT,
                                     (_NN - _N_USERS) - _UPT)
                pltpu.async_copy(
                    acc.at[pl.ds(_N_USERS + istart, _UPT), :],
                    items_h.at[pl.ds(istart, _UPT), h,
                               pl.ds(cx * _HALF, _HALF)], dsem[1])

            for d in descs:
                d.start()
                d.wait()
            pltpu.make_async_copy(
                acc.at[pl.ds(0, _UPT), :],
                users_h.at[pl.ds(0, _UPT), h, pl.ds(0, _HALF)],
                dsem[1]).wait()

            plsc.subcore_barrier()
            zero_acc()
            plsc.subcore_barrier()

        zero_acc()
        plsc.subcore_barrier()
        hop(table_h, 0)
        hop(stage_h, 1)
        hop(stage_h, 2)

    return prop_kernel(table, col2d, row2d, vals2d)


def kernel(user_embed, item_embed, edge_index, edge_vals):
    all_embed = jnp.concatenate([user_embed, item_embed], axis=0)
    table = jnp.stack([all_embed[:, :_HALF], all_embed[:, _HALF:]], axis=0)

    pad = _EPAD - _E
    colp = jnp.pad(edge_index[1], (0, pad)).reshape(_NROWS, _CHUNK)
    rowp = jnp.pad(edge_index[0], (0, pad)).reshape(_NROWS, _CHUNK)
    valsp = jnp.pad(edge_vals, (0, pad)).reshape(_NROWS, _CHUNK)

    users, items, _ = _propagate(table, colp, rowp, valsp)
    u = jnp.concatenate([user_embed[:, None, :], users], axis=1)
    i = jnp.concatenate([item_embed[:, None, :], items], axis=1)
    return u, i
